# Initial kernel scaffold; baseline (speedup 1.0000x reference)
import jax
import jax.numpy as jnp
from jax.experimental import pallas as pl


def kernel(last_lossalldata, now_lossalldata, quat, trans, sx, sq, beta, bindex, needgtloss, rotainput):
    raise NotImplementedError("write your pallas kernel here")


# speedup vs baseline: 2.1985x; 2.1985x over previous
import jax
import jax.numpy as jnp
from jax import lax
from jax.experimental import pallas as pl
from jax.experimental.pallas import tpu as pltpu

_B, _N = 4, 4096
_TQ = 512


def _nn_icp_body(lastT_ref, nowv_ref, rotaT_ref, trans3_ref, quat_ref,
                 transf_ref, beta_ref, out_ref):
    b = pl.program_id(0)
    q = pl.program_id(1)

    @pl.when((b == 0) & (q == 0))
    def _init():
        quat = quat_ref[...]
        tr = transf_ref[...]
        beta = beta_ref[0, 0]
        dx = tr - jnp.clip(tr, -10.0, 10.0)
        loss_x = jnp.sum(dx * dx) * (1.0 / (_B * 3))
        dq1 = quat[:, :2, :] - jnp.clip(quat[:, :2, :], -15.0, 15.0)
        loss_q1 = jnp.sum(dq1 * dq1) * (1.0 / (_B * 2 * 3))
        dq2 = quat[:, 2, :] - jnp.clip(quat[:, 2, :], -15.0, 15.0)
        loss_q2 = jnp.sum(dq2 * dq2) * (1.0 / (_B * 3))
        out_ref[...] = (loss_x + (loss_q1 + loss_q2) * beta).reshape(1, 1)

    vl = lastT_ref[0, 0:3, :]
    nl = lastT_ref[0, 3:6, :]
    vm = nowv_ref[0]
    rotaT = rotaT_ref[0]
    tr_b = trans3_ref[0]

    p = jnp.dot(vm, rotaT, preferred_element_type=jnp.float32) + tr_b
    pv = jnp.dot(p, vl, preferred_element_type=jnp.float32)
    pn = jnp.dot(p, nl, preferred_element_type=jnp.float32)
    v2 = jnp.sum(vl * vl, axis=0, keepdims=True)
    c = jnp.sum(vl * nl, axis=0, keepdims=True)

    dmat = v2 - 2.0 * pv
    rmat = pn - c
    m = jnp.min(dmat, axis=1, keepdims=True)
    iota = lax.broadcasted_iota(jnp.int32, dmat.shape, 1)
    jstar = jnp.min(jnp.where(dmat == m, iota, _N), axis=1, keepdims=True)
    r = jnp.sum(jnp.where(iota == jstar, rmat, 0.0), axis=1)
    out_ref[...] += jnp.sum(jnp.abs(r)).reshape(1, 1)


def kernel(last_lossalldata, now_lossalldata, quat, trans, sx, sq, beta,
           bindex, needgtloss, rotainput):
    lastT = jnp.transpose(last_lossalldata, (0, 2, 1))
    nowv = now_lossalldata[:, :, :3]
    rotaT = jnp.transpose(quat, (0, 2, 1))
    trans3 = trans[:, None, :]
    beta2 = beta.reshape(1, 1)

    nq = _N // _TQ
    out = pl.pallas_call(
        _nn_icp_body,
        grid=(_B, nq),
        in_specs=[
            pl.BlockSpec((1, 6, _N), lambda b, q: (b, 0, 0)),
            pl.BlockSpec((1, _TQ, 3), lambda b, q: (b, q, 0)),
            pl.BlockSpec((1, 3, 3), lambda b, q: (b, 0, 0)),
            pl.BlockSpec((1, 1, 3), lambda b, q: (b, 0, 0)),
            pl.BlockSpec((_B, 3, 3), lambda b, q: (0, 0, 0)),
            pl.BlockSpec((_B, 3), lambda b, q: (0, 0)),
            pl.BlockSpec((1, 1), lambda b, q: (0, 0)),
        ],
        out_specs=pl.BlockSpec((1, 1), lambda b, q: (0, 0)),
        out_shape=jax.ShapeDtypeStruct((1, 1), jnp.float32),
        compiler_params=pltpu.CompilerParams(
            dimension_semantics=("arbitrary", "arbitrary")),
    )(lastT, nowv, rotaT, trans3, quat, trans, beta2)
    return out[0, 0]


# speedup vs baseline: 3.5076x; 1.5955x over previous
import jax
import jax.numpy as jnp
from jax.experimental import pallas as pl
from jax.experimental.pallas import tpu as pltpu

_B, _N = 4, 4096
_TQ = 1024
_CK = 1024


def _nn_icp_body(lastT_ref, nowv_ref, rotaT_ref, trans3_ref, quat_ref,
                 transf_ref, beta_ref, out_ref, kd_ref, kr_ref):
    b = pl.program_id(0)
    q = pl.program_id(1)

    @pl.when((b == 0) & (q == 0))
    def _init():
        quat = quat_ref[...]
        tr = transf_ref[...]
        beta = beta_ref[0, 0]
        dx = tr - jnp.clip(tr, -10.0, 10.0)
        loss_x = jnp.sum(dx * dx) * (1.0 / (_B * 3))
        dq1 = quat[:, :2, :] - jnp.clip(quat[:, :2, :], -15.0, 15.0)
        loss_q1 = jnp.sum(dq1 * dq1) * (1.0 / (_B * 2 * 3))
        dq2 = quat[:, 2, :] - jnp.clip(quat[:, 2, :], -15.0, 15.0)
        loss_q2 = jnp.sum(dq2 * dq2) * (1.0 / (_B * 3))
        out_ref[...] = (loss_x + (loss_q1 + loss_q2) * beta).reshape(1, 1)

    @pl.when(q == 0)
    def _build_keys():
        vl = lastT_ref[0, 0:3, :]
        nl = lastT_ref[0, 3:6, :]
        kd_ref[0:3, :] = -2.0 * vl
        kd_ref[3:4, :] = jnp.sum(vl * vl, axis=0, keepdims=True)
        kr_ref[0:3, :] = nl
        kr_ref[3:4, :] = -jnp.sum(vl * nl, axis=0, keepdims=True)

    vm = nowv_ref[0]
    p = jnp.dot(vm, rotaT_ref[0], preferred_element_type=jnp.float32)
    p = p + trans3_ref[0]
    paug = jnp.concatenate([p, jnp.ones((_TQ, 1), jnp.float32)], axis=1)

    dmat = jnp.dot(paug, kd_ref[...], preferred_element_type=jnp.float32)
    rmat = jnp.dot(paug, kr_ref[...], preferred_element_type=jnp.float32)
    m = jnp.min(dmat, axis=1, keepdims=True)
    r = jnp.sum(jnp.where(dmat == m, rmat, 0.0), axis=1)
    out_ref[...] += jnp.sum(jnp.abs(r)).reshape(1, 1)


def kernel(last_lossalldata, now_lossalldata, quat, trans, sx, sq, beta,
           bindex, needgtloss, rotainput):
    lastT = jnp.transpose(last_lossalldata, (0, 2, 1))
    nowv = now_lossalldata[:, :, :3]
    rotaT = jnp.transpose(quat, (0, 2, 1))
    trans3 = trans[:, None, :]
    beta2 = beta.reshape(1, 1)

    nq = _N // _TQ
    out = pl.pallas_call(
        _nn_icp_body,
        grid=(_B, nq),
        in_specs=[
            pl.BlockSpec((1, 6, _N), lambda b, q: (b, 0, 0)),
            pl.BlockSpec((1, _TQ, 3), lambda b, q: (b, q, 0)),
            pl.BlockSpec((1, 3, 3), lambda b, q: (b, 0, 0)),
            pl.BlockSpec((1, 1, 3), lambda b, q: (b, 0, 0)),
            pl.BlockSpec((_B, 3, 3), lambda b, q: (0, 0, 0)),
            pl.BlockSpec((_B, 3), lambda b, q: (0, 0)),
            pl.BlockSpec((1, 1), lambda b, q: (0, 0)),
        ],
        out_specs=pl.BlockSpec((1, 1), lambda b, q: (0, 0)),
        out_shape=jax.ShapeDtypeStruct((1, 1), jnp.float32),
        scratch_shapes=[
            pltpu.VMEM((4, _N), jnp.float32),
            pltpu.VMEM((4, _N), jnp.float32),
        ],
        compiler_params=pltpu.CompilerParams(
            dimension_semantics=("arbitrary", "arbitrary")),
    )(lastT, nowv, rotaT, trans3, quat, trans, beta2)
    return out[0, 0]


# speedup vs baseline: 3.6525x; 1.0413x over previous
import jax
import jax.numpy as jnp
from jax.experimental import pallas as pl
from jax.experimental.pallas import tpu as pltpu

_B, _N = 4, 4096
_TQ = 1024


def _nn_icp_body(lastT_ref, nowv_ref, rotaT_ref, trans3_ref, quat_ref,
                 transf_ref, beta_ref, out_ref, kd_ref, kr_ref):
    b = pl.program_id(0)

    @pl.when(b == 0)
    def _init():
        quat = quat_ref[...]
        tr = transf_ref[...]
        beta = beta_ref[0, 0]
        dx = tr - jnp.clip(tr, -10.0, 10.0)
        loss_x = jnp.sum(dx * dx) * (1.0 / (_B * 3))
        dq1 = quat[:, :2, :] - jnp.clip(quat[:, :2, :], -15.0, 15.0)
        loss_q1 = jnp.sum(dq1 * dq1) * (1.0 / (_B * 2 * 3))
        dq2 = quat[:, 2, :] - jnp.clip(quat[:, 2, :], -15.0, 15.0)
        loss_q2 = jnp.sum(dq2 * dq2) * (1.0 / (_B * 3))
        out_ref[...] = (loss_x + (loss_q1 + loss_q2) * beta).reshape(1, 1)

    vl = lastT_ref[0, 0:3, :]
    nl = lastT_ref[0, 3:6, :]
    kd_ref[0:3, :] = -2.0 * vl
    kd_ref[3:4, :] = jnp.sum(vl * vl, axis=0, keepdims=True)
    kr_ref[0:3, :] = nl
    kr_ref[3:4, :] = -jnp.sum(vl * nl, axis=0, keepdims=True)
    kd = kd_ref[...]
    kr = kr_ref[...].astype(jnp.bfloat16)

    acc = jnp.zeros((), jnp.float32)
    for q in range(_N // _TQ):
        vm = nowv_ref[0, q * _TQ:(q + 1) * _TQ, :]
        p = jnp.dot(vm, rotaT_ref[0], preferred_element_type=jnp.float32)
        p = p + trans3_ref[0]
        paug = jnp.concatenate([p, jnp.ones((_TQ, 1), jnp.float32)], axis=1)

        dmat = jnp.dot(paug, kd, preferred_element_type=jnp.float32)
        rmat = jnp.dot(paug.astype(jnp.bfloat16), kr,
                       preferred_element_type=jnp.float32)
        mrun = jnp.full((_TQ, 128), jnp.inf, jnp.float32)
        rrun = jnp.zeros((_TQ, 128), jnp.float32)
        for c in range(0, _N, 128):
            dc = dmat[:, c:c + 128]
            rc = rmat[:, c:c + 128]
            mask = dc < mrun
            rrun = jnp.where(mask, rc, rrun)
            mrun = jnp.minimum(mrun, dc)
        mf = jnp.min(mrun, axis=1, keepdims=True)
        r = jnp.sum(jnp.where(mrun == mf, rrun, 0.0), axis=1)
        acc += jnp.sum(jnp.abs(r))
    out_ref[...] += acc.reshape(1, 1)


def kernel(last_lossalldata, now_lossalldata, quat, trans, sx, sq, beta,
           bindex, needgtloss, rotainput):
    lastT = jnp.transpose(last_lossalldata, (0, 2, 1))
    nowv = now_lossalldata[:, :, :3]
    rotaT = jnp.transpose(quat, (0, 2, 1))
    trans3 = trans[:, None, :]
    beta2 = beta.reshape(1, 1)

    out = pl.pallas_call(
        _nn_icp_body,
        grid=(_B,),
        in_specs=[
            pl.BlockSpec((1, 6, _N), lambda b: (b, 0, 0)),
            pl.BlockSpec((1, _N, 3), lambda b: (b, 0, 0)),
            pl.BlockSpec((1, 3, 3), lambda b: (b, 0, 0)),
            pl.BlockSpec((1, 1, 3), lambda b: (b, 0, 0)),
            pl.BlockSpec((_B, 3, 3), lambda b: (0, 0, 0)),
            pl.BlockSpec((_B, 3), lambda b: (0, 0)),
            pl.BlockSpec((1, 1), lambda b: (0, 0)),
        ],
        out_specs=pl.BlockSpec((1, 1), lambda b: (0, 0)),
        out_shape=jax.ShapeDtypeStruct((1, 1), jnp.float32),
        scratch_shapes=[
            pltpu.VMEM((4, _N), jnp.float32),
            pltpu.VMEM((4, _N), jnp.float32),
        ],
        compiler_params=pltpu.CompilerParams(
            dimension_semantics=("arbitrary",)),
    )(lastT, nowv, rotaT, trans3, quat, trans, beta2)
    return out[0, 0]
